# baseline (speedup 1.0000x reference)
import functools
import math

import jax
import jax.numpy as jnp
from jax.experimental import pallas as pl
from jax.experimental.pallas import tpu as pltpu

_HI = jax.lax.Precision.HIGHEST


def _mean_proj_krn(h_ref, w1_ref, o_ref, *, BB):
    x = h_ref[...]
    TP = x.shape[-1]
    p = ((x[..., 0:8] + x[..., 8:16]) + x[..., 16:24]) + x[..., 24:32]
    q = p[..., 0:4] + p[..., 4:8]
    r = q[..., 0:2] + q[..., 2:4]
    x0 = (r[..., 0] + r[..., 1]) * (1.0 / TP)
    w1 = w1_ref[...].astype(jnp.bfloat16)
    for i in range(BB):
        o_ref[i] = jnp.dot(x0[i].astype(jnp.bfloat16), w1,
                           preferred_element_type=jnp.float32)


def _graph_krn(src_ref, dst_ref, ew_ref, wd_ref, m1_ref, *, C):
    src = src_ref[0, :]
    dst = dst_ref[0, :]
    ew = ew_ref[0, :]
    E = src.shape[0]
    node = jax.lax.broadcasted_iota(jnp.int32, (E, C), 1)
    sh = jnp.where(src[:, None] == node, ew[:, None], 0.0)
    dh = (dst[:, None] == node).astype(jnp.float32)
    w = jax.lax.dot_general(dh, sh, (((0,), (0,)), ((), ())),
                            precision=_HI,
                            preferred_element_type=jnp.float32)
    deg = jnp.sum(w, axis=1) + 1.0
    dinv = jnp.where(deg > 0, deg ** -0.5, 0.0)
    eye = (jax.lax.broadcasted_iota(jnp.int32, (C, C), 0)
           == jax.lax.broadcasted_iota(jnp.int32, (C, C), 1)).astype(jnp.float32)
    m1 = dinv[:, None] * w * dinv[None, :] + eye * (dinv * dinv)[:, None]
    wd_ref[...] = w
    m1_ref[...] = m1


def _conv1_krn(xw_ref, m1_ref, b1_ref, y_ref, s_ref, *, TB):
    @pl.when(pl.program_id(0) == 0)
    def _init():
        s_ref[...] = jnp.zeros_like(s_ref)

    m1 = m1_ref[...]
    b1 = b1_ref[...]
    sm = jnp.zeros(b1.shape[1], jnp.float32)
    sq = jnp.zeros(b1.shape[1], jnp.float32)
    for i in range(TB):
        y = jnp.dot(m1, xw_ref[i], precision=_HI,
                    preferred_element_type=jnp.float32) + b1
        y_ref[i] = y
        sm = sm + jnp.sum(y, axis=0)
        sq = sq + jnp.sum(y * y, axis=0)
    pad = jnp.zeros((s_ref.shape[0] - 2, b1.shape[1]), jnp.float32)
    s_ref[...] += jnp.concatenate([sm[None, :], sq[None, :], pad], axis=0)


def _bn1_attn_krn(y_ref, s_ref, g_ref, be_ref, a_ref, m1_ref, wp_ref, bp_ref,
                  x1_ref, sc_ref, *, TB, N):
    mu = s_ref[0, :] / N
    var = s_ref[1, :] / N - mu * mu
    scale = g_ref[0, :] * jax.lax.rsqrt(var + 1e-5)
    shift = be_ref[0, :] - mu * scale
    a = a_ref[0, 0]
    m1 = m1_ref[...]
    wp = wp_ref[0, :].astype(jnp.bfloat16).astype(jnp.float32)
    bp = bp_ref[0, 0]
    for i in range(TB):
        xb = y_ref[i] * scale[None, :] + shift[None, :]
        xb = jnp.maximum(xb, 0.0) + a * jnp.minimum(xb, 0.0)
        x1_ref[i] = xb
        xbq = xb.astype(jnp.bfloat16).astype(jnp.float32)
        av = jnp.sum(xbq * wp[None, :], axis=1)
        attn = jnp.sum(m1 * av[None, :], axis=1) + bp
        sc_ref[i, :] = jnp.tanh(attn)


def _select_krn(sc_ref, m_ref, sv_ref, wm_ref, rk_ref, *, k):
    s = sc_ref[...]
    B, C = s.shape
    ci = jax.lax.broadcasted_iota(jnp.int32, (1, C, 1), 1)
    cj = jax.lax.broadcasted_iota(jnp.int32, (1, 1, C), 2)
    gt = s[:, None, :] > s[:, :, None]
    eq = (s[:, None, :] == s[:, :, None]) & (cj < ci)
    rank = jnp.sum((gt | eq).astype(jnp.float32), axis=2)
    kept = rank < float(k)
    m = kept.astype(jnp.float32)
    rowmax = jnp.max(s, axis=1, keepdims=True)
    e = jnp.exp(s - rowmax) * m
    wsum = jnp.sum(e, axis=1, keepdims=True)
    m_ref[...] = m
    sv_ref[...] = jnp.where(kept, s, 0.0)
    wm_ref[...] = e / wsum
    rk_ref[...] = rank


def _conv2_krn(x1_ref, sv_ref, m_ref, wd_ref, w2_ref, b2_ref, y_ref, s_ref,
               *, TB, C):
    @pl.when(pl.program_id(0) == 0)
    def _init():
        s_ref[...] = jnp.zeros_like(s_ref)

    wd = wd_ref[...]
    w2 = w2_ref[...]
    b2 = b2_ref[...]
    eye = (jax.lax.broadcasted_iota(jnp.int32, (C, C), 0)
           == jax.lax.broadcasted_iota(jnp.int32, (C, C), 1)).astype(jnp.float32)
    nout = b2.shape[1]
    sm = jnp.zeros(nout, jnp.float32)
    sq = jnp.zeros(nout, jnp.float32)
    w2b = w2.astype(jnp.bfloat16)
    for i in range(TB):
        xin = x1_ref[i] * sv_ref[i, :][:, None]
        xw = jnp.dot(xin.astype(jnp.bfloat16), w2b,
                     preferred_element_type=jnp.float32)
        mb = m_ref[i, :]
        wm = wd * mb[:, None] * mb[None, :]
        deg2 = jnp.sum(wm, axis=1) + mb
        dinv2 = jnp.where(deg2 > 0, deg2 ** -0.5, 0.0)
        m2 = dinv2[:, None] * wm * dinv2[None, :] + eye * (dinv2 * dinv2)[:, None]
        y = jnp.dot(m2, xw, precision=_HI,
                    preferred_element_type=jnp.float32) + b2
        y_ref[i] = y
        ym = mb[:, None] * y
        sm = sm + jnp.sum(ym, axis=0)
        sq = sq + jnp.sum(ym * y, axis=0)
    pad = jnp.zeros((s_ref.shape[0] - 2, nout), jnp.float32)
    s_ref[...] += jnp.concatenate([sm[None, :], sq[None, :], pad], axis=0)


def _bn2_out_krn(y_ref, s_ref, g_ref, be_ref, a_ref, rk_ref, m_ref, o_ref,
                 *, TB, N2):
    mu = s_ref[0, :] / N2
    var = s_ref[1, :] / N2 - mu * mu
    scale = g_ref[0, :] * jax.lax.rsqrt(var + 1e-5)
    shift = be_ref[0, :] - mu * scale
    a = a_ref[0, 0]
    C = y_ref.shape[1]
    jf = jax.lax.broadcasted_iota(jnp.int32, (C, C), 0).astype(jnp.float32)
    for i in range(TB):
        xb = y_ref[i] * scale[None, :] + shift[None, :]
        xb = jnp.maximum(xb, 0.0) + a * jnp.minimum(xb, 0.0)
        p = jnp.where((rk_ref[i, :][None, :] == jf) & (m_ref[i, :][None, :] > 0),
                      1.0, 0.0)
        o_ref[i] = jnp.dot(p, xb, precision=_HI,
                           preferred_element_type=jnp.float32)


def _z_krn(h_ref, wm_ref, wp_ref, bp_ref, z_ref, *, ZB):
    zp = jnp.sum(h_ref[...] * wm_ref[...][:, 0, :, None, None], axis=1)
    wp = wp_ref[...]
    bp = bp_ref[0, :]
    for i in range(ZB):
        z = jax.lax.dot_general(wp, zp[i], (((0,), (0,)), ((), ())),
                                precision=_HI,
                                preferred_element_type=jnp.float32)
        z_ref[i] = z + bp[:, None]


def kernel(h, edge_index, edge_weight, W_proj, b_proj, W1, b1, g1, be1, a1,
           Wp, bp, W2, b2, g2, be2, a2):
    B, C, F, TP = h.shape
    NHID = W1.shape[1]
    NOUT = W2.shape[1]
    E = edge_index.shape[1]
    N = B * C
    k = int(math.ceil(0.6 * C))
    N2 = B * k
    f32 = jnp.float32

    ei = edge_index.astype(jnp.int32)
    src = ei[0].reshape(1, E)
    dst = ei[1].reshape(1, E)
    ew = edge_weight.reshape(1, E)

    BB = 2
    xw1 = pl.pallas_call(
        functools.partial(_mean_proj_krn, BB=BB),
        grid=(B // BB,),
        in_specs=[pl.BlockSpec((BB, C, F, TP), lambda i: (i, 0, 0, 0)),
                  pl.BlockSpec((F, NHID), lambda i: (0, 0))],
        out_specs=pl.BlockSpec((BB, C, NHID), lambda i: (i, 0, 0)),
        out_shape=jax.ShapeDtypeStruct((B, C, NHID), f32),
    )(h, W1)

    wd, m1 = pl.pallas_call(
        functools.partial(_graph_krn, C=C),
        in_specs=[pl.BlockSpec((1, E), lambda: (0, 0))] * 3,
        out_specs=[pl.BlockSpec((C, C), lambda: (0, 0))] * 2,
        out_shape=[jax.ShapeDtypeStruct((C, C), f32)] * 2,
    )(src, dst, ew)

    TB = 8
    acc = pltpu.CompilerParams(dimension_semantics=("arbitrary",))
    y1, s1 = pl.pallas_call(
        functools.partial(_conv1_krn, TB=TB),
        grid=(B // TB,),
        in_specs=[pl.BlockSpec((TB, C, NHID), lambda i: (i, 0, 0)),
                  pl.BlockSpec((C, C), lambda i: (0, 0)),
                  pl.BlockSpec((1, NHID), lambda i: (0, 0))],
        out_specs=[pl.BlockSpec((TB, C, NHID), lambda i: (i, 0, 0)),
                   pl.BlockSpec((8, NHID), lambda i: (0, 0))],
        out_shape=[jax.ShapeDtypeStruct((B, C, NHID), f32),
                   jax.ShapeDtypeStruct((8, NHID), f32)],
        compiler_params=acc,
    )(xw1, m1, b1.reshape(1, NHID))

    x1, score = pl.pallas_call(
        functools.partial(_bn1_attn_krn, TB=TB, N=N),
        grid=(B // TB,),
        in_specs=[pl.BlockSpec((TB, C, NHID), lambda i: (i, 0, 0)),
                  pl.BlockSpec((8, NHID), lambda i: (0, 0)),
                  pl.BlockSpec((1, NHID), lambda i: (0, 0)),
                  pl.BlockSpec((1, NHID), lambda i: (0, 0)),
                  pl.BlockSpec((1, 1), lambda i: (0, 0)),
                  pl.BlockSpec((C, C), lambda i: (0, 0)),
                  pl.BlockSpec((1, NHID), lambda i: (0, 0)),
                  pl.BlockSpec((1, 1), lambda i: (0, 0))],
        out_specs=[pl.BlockSpec((TB, C, NHID), lambda i: (i, 0, 0)),
                   pl.BlockSpec((TB, C), lambda i: (i, 0))],
        out_shape=[jax.ShapeDtypeStruct((B, C, NHID), f32),
                   jax.ShapeDtypeStruct((B, C), f32)],
    )(y1, s1, g1.reshape(1, NHID), be1.reshape(1, NHID),
      a1.reshape(1, 1), m1, Wp.reshape(1, NHID), bp.reshape(1, 1))

    m, sv, wm, rk = pl.pallas_call(
        functools.partial(_select_krn, k=k),
        in_specs=[pl.BlockSpec((B, C), lambda: (0, 0))],
        out_specs=[pl.BlockSpec((B, C), lambda: (0, 0))] * 4,
        out_shape=[jax.ShapeDtypeStruct((B, C), f32)] * 4,
    )(score)

    y2, s2 = pl.pallas_call(
        functools.partial(_conv2_krn, TB=TB, C=C),
        grid=(B // TB,),
        in_specs=[pl.BlockSpec((TB, C, NHID), lambda i: (i, 0, 0)),
                  pl.BlockSpec((TB, C), lambda i: (i, 0)),
                  pl.BlockSpec((TB, C), lambda i: (i, 0)),
                  pl.BlockSpec((C, C), lambda i: (0, 0)),
                  pl.BlockSpec((NHID, NOUT), lambda i: (0, 0)),
                  pl.BlockSpec((1, NOUT), lambda i: (0, 0))],
        out_specs=[pl.BlockSpec((TB, C, NOUT), lambda i: (i, 0, 0)),
                   pl.BlockSpec((8, NOUT), lambda i: (0, 0))],
        out_shape=[jax.ShapeDtypeStruct((B, C, NOUT), f32),
                   jax.ShapeDtypeStruct((8, NOUT), f32)],
        compiler_params=acc,
    )(x1, sv, m, wd, W2, b2.reshape(1, NOUT))

    xo = pl.pallas_call(
        functools.partial(_bn2_out_krn, TB=TB, N2=N2),
        grid=(B // TB,),
        in_specs=[pl.BlockSpec((TB, C, NOUT), lambda i: (i, 0, 0)),
                  pl.BlockSpec((8, NOUT), lambda i: (0, 0)),
                  pl.BlockSpec((1, NOUT), lambda i: (0, 0)),
                  pl.BlockSpec((1, NOUT), lambda i: (0, 0)),
                  pl.BlockSpec((1, 1), lambda i: (0, 0)),
                  pl.BlockSpec((TB, C), lambda i: (i, 0)),
                  pl.BlockSpec((TB, C), lambda i: (i, 0))],
        out_specs=pl.BlockSpec((TB, C, NOUT), lambda i: (i, 0, 0)),
        out_shape=jax.ShapeDtypeStruct((B, C, NOUT), f32),
    )(y2, s2, g2.reshape(1, NOUT), be2.reshape(1, NOUT), a2.reshape(1, 1),
      rk, m)
    x_out = xo[:, :k, :].reshape(N2, NOUT)

    ZB = 2
    z_seq = pl.pallas_call(
        functools.partial(_z_krn, ZB=ZB),
        grid=(B // ZB,),
        in_specs=[pl.BlockSpec((ZB, C, F, TP), lambda i: (i, 0, 0, 0)),
                  pl.BlockSpec((ZB, 1, C), lambda i: (i, 0, 0)),
                  pl.BlockSpec((F, NOUT), lambda i: (0, 0)),
                  pl.BlockSpec((1, NOUT), lambda i: (0, 0))],
        out_specs=pl.BlockSpec((ZB, NOUT, TP), lambda i: (i, 0, 0)),
        out_shape=jax.ShapeDtypeStruct((B, NOUT, TP), f32),
    )(h, wm.reshape(B, 1, C), W_proj, b_proj.reshape(1, NOUT))

    return x_out, z_seq


# speedup vs baseline: 1.1171x; 1.1171x over previous
import functools
import math

import jax
import jax.numpy as jnp
from jax.experimental import pallas as pl
from jax.experimental.pallas import tpu as pltpu

_HI = jax.lax.Precision.HIGHEST


def _mean_proj_krn(h_ref, w1_ref, o_ref):
    x = h_ref[...]
    TP = x.shape[-1]
    p = ((x[..., 0:8] + x[..., 8:16]) + x[..., 16:24]) + x[..., 24:32]
    q = p[..., 0:4] + p[..., 4:8]
    r = q[..., 0:2] + q[..., 2:4]
    x0 = (r[..., 0] + r[..., 1]) * (1.0 / TP)
    o_ref[...] = jnp.dot(x0.astype(jnp.bfloat16),
                         w1_ref[...].astype(jnp.bfloat16),
                         preferred_element_type=jnp.float32)


def _graph_krn(src_ref, dst_ref, ew_ref, wd_ref, m1_ref, *, C):
    src = src_ref[0, :]
    dst = dst_ref[0, :]
    ew = ew_ref[0, :]
    E = src.shape[0]
    node = jax.lax.broadcasted_iota(jnp.int32, (E, C), 1)
    sh = jnp.where(src[:, None] == node, ew[:, None], 0.0)
    dh = (dst[:, None] == node).astype(jnp.float32)
    w = jax.lax.dot_general(dh, sh, (((0,), (0,)), ((), ())),
                            precision=_HI,
                            preferred_element_type=jnp.float32)
    deg = jnp.sum(w, axis=1) + 1.0
    dinv = jnp.where(deg > 0, deg ** -0.5, 0.0)
    eye = (jax.lax.broadcasted_iota(jnp.int32, (C, C), 0)
           == jax.lax.broadcasted_iota(jnp.int32, (C, C), 1)).astype(jnp.float32)
    m1 = dinv[:, None] * w * dinv[None, :] + eye * (dinv * dinv)[:, None]
    wd_ref[...] = w
    m1_ref[...] = m1


def _conv1_krn(xw_ref, m1_ref, b1_ref, y_ref, s_ref, *, TB):
    @pl.when(pl.program_id(0) == 0)
    def _init():
        s_ref[...] = jnp.zeros_like(s_ref)

    m1 = m1_ref[...]
    b1 = b1_ref[...]
    sm = jnp.zeros(b1.shape[1], jnp.float32)
    sq = jnp.zeros(b1.shape[1], jnp.float32)
    for i in range(TB):
        y = jnp.dot(m1, xw_ref[i], precision=_HI,
                    preferred_element_type=jnp.float32) + b1
        y_ref[i] = y
        sm = sm + jnp.sum(y, axis=0)
        sq = sq + jnp.sum(y * y, axis=0)
    pad = jnp.zeros((s_ref.shape[0] - 2, b1.shape[1]), jnp.float32)
    s_ref[...] += jnp.concatenate([sm[None, :], sq[None, :], pad], axis=0)


def _bn1_attn_krn(y_ref, s_ref, g_ref, be_ref, a_ref, m1_ref, wp_ref, bp_ref,
                  x1_ref, sc_ref, *, TB, N):
    mu = s_ref[0, :] / N
    var = s_ref[1, :] / N - mu * mu
    scale = g_ref[0, :] * jax.lax.rsqrt(var + 1e-5)
    shift = be_ref[0, :] - mu * scale
    a = a_ref[0, 0]
    m1 = m1_ref[...]
    wp = wp_ref[0, :].astype(jnp.bfloat16).astype(jnp.float32)
    bp = bp_ref[0, 0]
    for i in range(TB):
        xb = y_ref[i] * scale[None, :] + shift[None, :]
        xb = jnp.maximum(xb, 0.0) + a * jnp.minimum(xb, 0.0)
        x1_ref[i] = xb
        xbq = xb.astype(jnp.bfloat16).astype(jnp.float32)
        av = jnp.sum(xbq * wp[None, :], axis=1)
        attn = jnp.sum(m1 * av[None, :], axis=1) + bp
        sc_ref[i, :] = jnp.tanh(attn)


def _select_krn(sc_ref, m_ref, sv_ref, wm_ref, rk_ref, *, k):
    s = sc_ref[...]
    B, C = s.shape
    ci = jax.lax.broadcasted_iota(jnp.int32, (1, C, 1), 1)
    cj = jax.lax.broadcasted_iota(jnp.int32, (1, 1, C), 2)
    gt = s[:, None, :] > s[:, :, None]
    eq = (s[:, None, :] == s[:, :, None]) & (cj < ci)
    rank = jnp.sum((gt | eq).astype(jnp.float32), axis=2)
    kept = rank < float(k)
    m = kept.astype(jnp.float32)
    rowmax = jnp.max(s, axis=1, keepdims=True)
    e = jnp.exp(s - rowmax) * m
    wsum = jnp.sum(e, axis=1, keepdims=True)
    m_ref[...] = m
    sv_ref[...] = jnp.where(kept, s, 0.0)
    wm_ref[...] = e / wsum
    rk_ref[...] = rank


def _conv2_krn(x1_ref, sv_ref, m_ref, wd_ref, w2_ref, b2_ref, y_ref, s_ref,
               *, TB, C):
    @pl.when(pl.program_id(0) == 0)
    def _init():
        s_ref[...] = jnp.zeros_like(s_ref)

    wd = wd_ref[...]
    w2 = w2_ref[...]
    b2 = b2_ref[...]
    eye = (jax.lax.broadcasted_iota(jnp.int32, (C, C), 0)
           == jax.lax.broadcasted_iota(jnp.int32, (C, C), 1)).astype(jnp.float32)
    nout = b2.shape[1]
    sm = jnp.zeros(nout, jnp.float32)
    sq = jnp.zeros(nout, jnp.float32)
    w2b = w2.astype(jnp.bfloat16)
    for i in range(TB):
        xin = x1_ref[i] * sv_ref[i, :][:, None]
        xw = jnp.dot(xin.astype(jnp.bfloat16), w2b,
                     preferred_element_type=jnp.float32)
        mb = m_ref[i, :]
        wm = wd * mb[:, None] * mb[None, :]
        deg2 = jnp.sum(wm, axis=1) + mb
        dinv2 = jnp.where(deg2 > 0, deg2 ** -0.5, 0.0)
        m2 = dinv2[:, None] * wm * dinv2[None, :] + eye * (dinv2 * dinv2)[:, None]
        y = jnp.dot(m2, xw, precision=_HI,
                    preferred_element_type=jnp.float32) + b2
        y_ref[i] = y
        ym = mb[:, None] * y
        sm = sm + jnp.sum(ym, axis=0)
        sq = sq + jnp.sum(ym * y, axis=0)
    pad = jnp.zeros((s_ref.shape[0] - 2, nout), jnp.float32)
    s_ref[...] += jnp.concatenate([sm[None, :], sq[None, :], pad], axis=0)


def _bn2_out_krn(y_ref, s_ref, g_ref, be_ref, a_ref, rk_ref, m_ref, o_ref,
                 *, TB, N2):
    mu = s_ref[0, :] / N2
    var = s_ref[1, :] / N2 - mu * mu
    scale = g_ref[0, :] * jax.lax.rsqrt(var + 1e-5)
    shift = be_ref[0, :] - mu * scale
    a = a_ref[0, 0]
    C = y_ref.shape[1]
    jf = jax.lax.broadcasted_iota(jnp.int32, (C, C), 0).astype(jnp.float32)
    for i in range(TB):
        xb = y_ref[i] * scale[None, :] + shift[None, :]
        xb = jnp.maximum(xb, 0.0) + a * jnp.minimum(xb, 0.0)
        p = jnp.where((rk_ref[i, :][None, :] == jf) & (m_ref[i, :][None, :] > 0),
                      1.0, 0.0)
        o_ref[i] = jnp.dot(p, xb, precision=_HI,
                           preferred_element_type=jnp.float32)


def _zpre_krn(h_ref, wm_ref, zp_ref):
    zp_ref[...] = jnp.sum(h_ref[...] * wm_ref[...][:, :, None], axis=1)


def _zproj_krn(zp_ref, wp_ref, bp_ref, z_ref, *, TB):
    wp = wp_ref[...]
    bp = bp_ref[0, :]
    for i in range(TB):
        z = jax.lax.dot_general(wp, zp_ref[i], (((0,), (0,)), ((), ())),
                                precision=_HI,
                                preferred_element_type=jnp.float32)
        z_ref[i] = z + bp[:, None]


def kernel(h, edge_index, edge_weight, W_proj, b_proj, W1, b1, g1, be1, a1,
           Wp, bp, W2, b2, g2, be2, a2):
    B, C, F, TP = h.shape
    NHID = W1.shape[1]
    NOUT = W2.shape[1]
    E = edge_index.shape[1]
    N = B * C
    k = int(math.ceil(0.6 * C))
    N2 = B * k
    f32 = jnp.float32

    ei = edge_index.astype(jnp.int32)
    src = ei[0].reshape(1, E)
    dst = ei[1].reshape(1, E)
    ew = edge_weight.reshape(1, E)

    RB = 64
    xw1 = pl.pallas_call(
        _mean_proj_krn,
        grid=(N // RB,),
        in_specs=[pl.BlockSpec((RB, F, TP), lambda i: (i, 0, 0)),
                  pl.BlockSpec((F, NHID), lambda i: (0, 0))],
        out_specs=pl.BlockSpec((RB, NHID), lambda i: (i, 0)),
        out_shape=jax.ShapeDtypeStruct((N, NHID), f32),
    )(h.reshape(N, F, TP), W1)

    wd, m1 = pl.pallas_call(
        functools.partial(_graph_krn, C=C),
        in_specs=[pl.BlockSpec((1, E), lambda: (0, 0))] * 3,
        out_specs=[pl.BlockSpec((C, C), lambda: (0, 0))] * 2,
        out_shape=[jax.ShapeDtypeStruct((C, C), f32)] * 2,
    )(src, dst, ew)

    TB = 8
    acc = pltpu.CompilerParams(dimension_semantics=("arbitrary",))
    y1, s1 = pl.pallas_call(
        functools.partial(_conv1_krn, TB=TB),
        grid=(B // TB,),
        in_specs=[pl.BlockSpec((TB, C, NHID), lambda i: (i, 0, 0)),
                  pl.BlockSpec((C, C), lambda i: (0, 0)),
                  pl.BlockSpec((1, NHID), lambda i: (0, 0))],
        out_specs=[pl.BlockSpec((TB, C, NHID), lambda i: (i, 0, 0)),
                   pl.BlockSpec((8, NHID), lambda i: (0, 0))],
        out_shape=[jax.ShapeDtypeStruct((B, C, NHID), f32),
                   jax.ShapeDtypeStruct((8, NHID), f32)],
        compiler_params=acc,
    )(xw1.reshape(B, C, NHID), m1, b1.reshape(1, NHID))

    x1, score = pl.pallas_call(
        functools.partial(_bn1_attn_krn, TB=TB, N=N),
        grid=(B // TB,),
        in_specs=[pl.BlockSpec((TB, C, NHID), lambda i: (i, 0, 0)),
                  pl.BlockSpec((8, NHID), lambda i: (0, 0)),
                  pl.BlockSpec((1, NHID), lambda i: (0, 0)),
                  pl.BlockSpec((1, NHID), lambda i: (0, 0)),
                  pl.BlockSpec((1, 1), lambda i: (0, 0)),
                  pl.BlockSpec((C, C), lambda i: (0, 0)),
                  pl.BlockSpec((1, NHID), lambda i: (0, 0)),
                  pl.BlockSpec((1, 1), lambda i: (0, 0))],
        out_specs=[pl.BlockSpec((TB, C, NHID), lambda i: (i, 0, 0)),
                   pl.BlockSpec((TB, C), lambda i: (i, 0))],
        out_shape=[jax.ShapeDtypeStruct((B, C, NHID), f32),
                   jax.ShapeDtypeStruct((B, C), f32)],
    )(y1, s1, g1.reshape(1, NHID), be1.reshape(1, NHID),
      a1.reshape(1, 1), m1, Wp.reshape(1, NHID), bp.reshape(1, 1))

    m, sv, wm, rk = pl.pallas_call(
        functools.partial(_select_krn, k=k),
        in_specs=[pl.BlockSpec((B, C), lambda: (0, 0))],
        out_specs=[pl.BlockSpec((B, C), lambda: (0, 0))] * 4,
        out_shape=[jax.ShapeDtypeStruct((B, C), f32)] * 4,
    )(score)

    y2, s2 = pl.pallas_call(
        functools.partial(_conv2_krn, TB=TB, C=C),
        grid=(B // TB,),
        in_specs=[pl.BlockSpec((TB, C, NHID), lambda i: (i, 0, 0)),
                  pl.BlockSpec((TB, C), lambda i: (i, 0)),
                  pl.BlockSpec((TB, C), lambda i: (i, 0)),
                  pl.BlockSpec((C, C), lambda i: (0, 0)),
                  pl.BlockSpec((NHID, NOUT), lambda i: (0, 0)),
                  pl.BlockSpec((1, NOUT), lambda i: (0, 0))],
        out_specs=[pl.BlockSpec((TB, C, NOUT), lambda i: (i, 0, 0)),
                   pl.BlockSpec((8, NOUT), lambda i: (0, 0))],
        out_shape=[jax.ShapeDtypeStruct((B, C, NOUT), f32),
                   jax.ShapeDtypeStruct((8, NOUT), f32)],
        compiler_params=acc,
    )(x1, sv, m, wd, W2, b2.reshape(1, NOUT))

    xo = pl.pallas_call(
        functools.partial(_bn2_out_krn, TB=TB, N2=N2),
        grid=(B // TB,),
        in_specs=[pl.BlockSpec((TB, C, NOUT), lambda i: (i, 0, 0)),
                  pl.BlockSpec((8, NOUT), lambda i: (0, 0)),
                  pl.BlockSpec((1, NOUT), lambda i: (0, 0)),
                  pl.BlockSpec((1, NOUT), lambda i: (0, 0)),
                  pl.BlockSpec((1, 1), lambda i: (0, 0)),
                  pl.BlockSpec((TB, C), lambda i: (i, 0)),
                  pl.BlockSpec((TB, C), lambda i: (i, 0))],
        out_specs=pl.BlockSpec((TB, C, NOUT), lambda i: (i, 0, 0)),
        out_shape=jax.ShapeDtypeStruct((B, C, NOUT), f32),
    )(y2, s2, g2.reshape(1, NOUT), be2.reshape(1, NOUT), a2.reshape(1, 1),
      rk, m)
    x_out = xo[:, :k, :].reshape(N2, NOUT)

    ZB = 8
    zp = pl.pallas_call(
        _zpre_krn,
        grid=(B // ZB,),
        in_specs=[pl.BlockSpec((ZB, C, F * TP), lambda i: (i, 0, 0)),
                  pl.BlockSpec((ZB, C), lambda i: (i, 0))],
        out_specs=pl.BlockSpec((ZB, F * TP), lambda i: (i, 0)),
        out_shape=jax.ShapeDtypeStruct((B, F * TP), f32),
    )(h.reshape(B, C, F * TP), wm)

    z_seq = pl.pallas_call(
        functools.partial(_zproj_krn, TB=ZB),
        grid=(B // ZB,),
        in_specs=[pl.BlockSpec((ZB, F, TP), lambda i: (i, 0, 0)),
                  pl.BlockSpec((F, NOUT), lambda i: (0, 0)),
                  pl.BlockSpec((1, NOUT), lambda i: (0, 0))],
        out_specs=pl.BlockSpec((ZB, NOUT, TP), lambda i: (i, 0, 0)),
        out_shape=jax.ShapeDtypeStruct((B, NOUT, TP), f32),
    )(zp.reshape(B, F, TP), W_proj, b_proj.reshape(1, NOUT))

    return x_out, z_seq
